# Initial kernel scaffold; baseline (speedup 1.0000x reference)
import jax
import jax.numpy as jnp
from jax.experimental import pallas as pl


def kernel(x, W1, b1, W2, b2, We, be):
    raise NotImplementedError("write your pallas kernel here")


# speedup vs baseline: 1.8154x; 1.8154x over previous
import functools

import jax
import jax.numpy as jnp
from jax.experimental import pallas as pl

MODEL_DIM = 768
OUT_DIM = 768
NUM_EXPERTS = 8
GATE_HIDDEN = 64
TOP_K = 2
N_TOKENS = 4096

BT = 512


def _moe_block(x_ref, w1_ref, b1_ref, w2_ref, b2_ref, we_ref, be_ref, out_ref):
    xb = x_ref[...]

    gx = jax.lax.dot_general(
        xb, w1_ref[...], (((1,), (1,)), ((), ())),
        preferred_element_type=jnp.float32)
    gx = jnp.maximum(gx + b1_ref[...], 0.0)
    logits = jax.lax.dot_general(
        gx, w2_ref[...], (((1,), (1,)), ((), ())),
        preferred_element_type=jnp.float32)
    logits = logits + b2_ref[...]

    m = jnp.max(logits, axis=1, keepdims=True)
    ex = jnp.exp(logits - m)
    w = ex / jnp.sum(ex, axis=1, keepdims=True)

    lane = jax.lax.broadcasted_iota(jnp.int32, w.shape, 1)
    m1 = jnp.max(w, axis=1, keepdims=True)
    is1 = w == m1
    i1 = jnp.min(jnp.where(is1, lane, NUM_EXPERTS), axis=1, keepdims=True)
    mask1 = lane == i1
    w_rest = jnp.where(mask1, -1.0, w)
    m2 = jnp.max(w_rest, axis=1, keepdims=True)
    is2 = w_rest == m2
    i2 = jnp.min(jnp.where(is2, lane, NUM_EXPERTS), axis=1, keepdims=True)
    mask2 = lane == i2
    denom = m1 + m2
    gating = (jnp.where(mask1, m1, 0.0) + jnp.where(mask2, m2, 0.0)) / denom

    acc = jnp.zeros((xb.shape[0], OUT_DIM), dtype=jnp.float32)
    for e in range(NUM_EXPERTS):
        y = jax.lax.dot_general(
            xb, we_ref[e], (((1,), (1,)), ((), ())),
            preferred_element_type=jnp.float32)
        y = jnp.maximum(y + be_ref[e][None, :], 0.0)
        acc = acc + gating[:, e][:, None] * y
    out_ref[...] = acc


@jax.jit
def kernel(x, W1, b1, W2, b2, We, be):
    n = x.shape[0]
    grid = (n // BT,)
    full = lambda shape: pl.BlockSpec(shape, lambda i: (0,) * len(shape))
    return pl.pallas_call(
        _moe_block,
        grid=grid,
        in_specs=[
            pl.BlockSpec((BT, MODEL_DIM), lambda i: (i, 0)),
            full((GATE_HIDDEN, MODEL_DIM)),
            full((1, GATE_HIDDEN)),
            full((NUM_EXPERTS, GATE_HIDDEN)),
            full((1, NUM_EXPERTS)),
            full((NUM_EXPERTS, OUT_DIM, MODEL_DIM)),
            full((NUM_EXPERTS, OUT_DIM)),
        ],
        out_specs=pl.BlockSpec((BT, OUT_DIM), lambda i: (i, 0)),
        out_shape=jax.ShapeDtypeStruct((n, OUT_DIM), jnp.float32),
    )(x, W1, b1.reshape(1, -1), W2, b2.reshape(1, -1), We, be)


# speedup vs baseline: 1.8407x; 1.0139x over previous
import functools

import jax
import jax.numpy as jnp
from jax.experimental import pallas as pl

MODEL_DIM = 768
OUT_DIM = 768
NUM_EXPERTS = 8
GATE_HIDDEN = 64
TOP_K = 2
N_TOKENS = 4096

BT = 512


def _moe_block(x_ref, w1_ref, b1_ref, w2_ref, b2_ref, we_ref, be_ref, out_ref):
    xb = x_ref[...]

    gx = jax.lax.dot_general(
        xb, w1_ref[...], (((1,), (1,)), ((), ())),
        preferred_element_type=jnp.float32)
    gx = jnp.maximum(gx + b1_ref[...], 0.0)
    logits = jax.lax.dot_general(
        gx, w2_ref[...], (((1,), (1,)), ((), ())),
        preferred_element_type=jnp.float32)
    logits = logits + b2_ref[...]

    m = jnp.max(logits, axis=1, keepdims=True)
    ex = jnp.exp(logits - m)
    w = ex / jnp.sum(ex, axis=1, keepdims=True)

    lane = jax.lax.broadcasted_iota(jnp.int32, w.shape, 1)
    m1 = jnp.max(w, axis=1, keepdims=True)
    is1 = w == m1
    i1 = jnp.min(jnp.where(is1, lane, NUM_EXPERTS), axis=1, keepdims=True)
    mask1 = lane == i1
    w_rest = jnp.where(mask1, -1.0, w)
    m2 = jnp.max(w_rest, axis=1, keepdims=True)
    is2 = w_rest == m2
    i2 = jnp.min(jnp.where(is2, lane, NUM_EXPERTS), axis=1, keepdims=True)
    mask2 = lane == i2
    denom = m1 + m2
    gating = (jnp.where(mask1, m1, 0.0) + jnp.where(mask2, m2, 0.0)) / denom

    xb16 = xb.astype(jnp.bfloat16)
    acc = jnp.zeros((xb.shape[0], OUT_DIM), dtype=jnp.float32)
    for e in range(NUM_EXPERTS):
        y = jax.lax.dot_general(
            xb16, we_ref[e].astype(jnp.bfloat16), (((1,), (1,)), ((), ())),
            preferred_element_type=jnp.float32)
        y = jnp.maximum(y + be_ref[e][None, :], 0.0)
        acc = acc + gating[:, e][:, None] * y
    out_ref[...] = acc


@jax.jit
def kernel(x, W1, b1, W2, b2, We, be):
    n = x.shape[0]
    grid = (n // BT,)
    full = lambda shape: pl.BlockSpec(shape, lambda i: (0,) * len(shape))
    return pl.pallas_call(
        _moe_block,
        grid=grid,
        in_specs=[
            pl.BlockSpec((BT, MODEL_DIM), lambda i: (i, 0)),
            full((GATE_HIDDEN, MODEL_DIM)),
            full((1, GATE_HIDDEN)),
            full((NUM_EXPERTS, GATE_HIDDEN)),
            full((1, NUM_EXPERTS)),
            full((NUM_EXPERTS, OUT_DIM, MODEL_DIM)),
            full((NUM_EXPERTS, OUT_DIM)),
        ],
        out_specs=pl.BlockSpec((BT, OUT_DIM), lambda i: (i, 0)),
        out_shape=jax.ShapeDtypeStruct((n, OUT_DIM), jnp.float32),
    )(x, W1, b1.reshape(1, -1), W2, b2.reshape(1, -1), We, be)
